# baseline (device time: 104498 ns/iter reference)
import jax
import jax.numpy as jnp
from jax import lax
from jax.experimental import pallas as pl
from jax.experimental.pallas import tpu as pltpu

N_DEV = 4
M_PER = 1024
H = 512
RS = 256
K = 4096
KT = 512
N_PER = 2048
N_STAGE = 3


def kernel(x, w_mat, scale_x, scale_w):
    def body(x_hbm, w_hbm, sx_ref, sw_ref, out_ref,
             x8_ref, xstage_ref, w_ref, wtile_ref,
             cw_ref, ccw_ref, stage_ref,
             xdma_sems, wdma_sems, s_cw, r_cw, s_ccw, r_ccw, copy_sems):
        my_pos = lax.axis_index("i")
        right = lax.rem(my_pos + 1, N_DEV)
        left = lax.rem(my_pos + N_DEV - 1, N_DEV)

        barrier_sem = pltpu.get_barrier_semaphore()
        for nbr in (left, right):
            pl.semaphore_signal(
                barrier_sem, inc=1,
                device_id=(nbr,), device_id_type=pl.DeviceIdType.MESH,
            )
        pl.semaphore_wait(barrier_sem, 2)

        scale = sx_ref[0] * sw_ref[0]
        cps = [None] * N_STAGE
        gemm_n = [0]
        sends = []

        def mk(buf, sems_s, sems_r, src, hop, sub, dev):
            return pltpu.make_async_remote_copy(
                src_ref=src,
                dst_ref=buf.at[hop, pl.ds(sub * RS, RS), :],
                send_sem=sems_s.at[hop, sub],
                recv_sem=sems_r.at[hop, sub],
                device_id=(dev,), device_id_type=pl.DeviceIdType.MESH,
            )

        def send_cw(src, hop, sub):
            r = mk(cw_ref, s_cw, r_cw, src, hop, sub, right)
            r.start()
            sends.append(r)

        def send_ccw(src, hop, sub):
            r = mk(ccw_ref, s_ccw, r_ccw, src, hop, sub, left)
            r.start()
            sends.append(r)

        def recv_cw(hop, sub):
            dst = cw_ref.at[hop, pl.ds(sub * RS, RS), :]
            pltpu.make_async_remote_copy(
                src_ref=dst, dst_ref=dst,
                send_sem=s_cw.at[hop, sub], recv_sem=r_cw.at[hop, sub],
                device_id=(right,), device_id_type=pl.DeviceIdType.MESH,
            ).wait_recv()

        def recv_ccw(hop, sub):
            dst = ccw_ref.at[hop, pl.ds(sub * RS, RS), :]
            pltpu.make_async_remote_copy(
                src_ref=dst, dst_ref=dst,
                send_sem=s_ccw.at[hop, sub], recv_sem=r_ccw.at[hop, sub],
                device_id=(left,), device_id_type=pl.DeviceIdType.MESH,
            ).wait_recv()

        def gemm_part(origin, row_off, chunk, nrows):
            slot = gemm_n[0] % N_STAGE
            gemm_n[0] += 1
            if cps[slot] is not None:
                cps[slot].wait()
            acc = jnp.dot(chunk, w_ref[...], preferred_element_type=jnp.float32)
            y = acc * scale
            stage_ref[slot, pl.ds(0, nrows), :] = y * jax.nn.sigmoid(y)
            cp = pltpu.make_async_copy(
                stage_ref.at[slot, pl.ds(0, nrows), :],
                out_ref.at[pl.ds(origin * M_PER + row_off, nrows), :],
                copy_sems.at[slot],
            )
            cp.start()
            cps[slot] = cp

        def gemm_half(origin, row_off, chunk):
            gemm_part(origin, row_off, chunk, H)

        sub_rows = (0, H, RS, H + RS)
        xd = []
        for i in range(2):
            d = pltpu.make_async_copy(
                x_hbm.at[pl.ds(sub_rows[i], RS), :],
                xstage_ref.at[i % 2], xdma_sems.at[i % 2])
            d.start()
            xd.append(d)
        for i in range(4):
            xd[i].wait()
            r0 = sub_rows[i]
            x8_ref[pl.ds(r0, RS), :] = xstage_ref[i % 2].astype(jnp.float8_e5m2)
            if i + 2 < 4:
                d = pltpu.make_async_copy(
                    x_hbm.at[pl.ds(sub_rows[i + 2], RS), :],
                    xstage_ref.at[i % 2], xdma_sems.at[i % 2])
                d.start()
                xd.append(d)
            if r0 < H:
                send_cw(x8_ref.at[pl.ds(r0, RS), :], 0, r0 // RS)
            else:
                send_ccw(x8_ref.at[pl.ds(r0, RS), :], 0, (r0 - H) // RS)

        def w_tiles(t0, t1):
            wd = []
            for t in range(t0, min(t0 + 2, t1)):
                d = pltpu.make_async_copy(
                    w_hbm.at[pl.ds(t * KT, KT), pl.ds(my_pos * N_PER, N_PER)],
                    wtile_ref.at[t % 2], wdma_sems.at[t % 2])
                d.start()
                wd.append((t, d))
            for t in range(t0, t1):
                wd[0][1].wait()
                wd.pop(0)
                if t + 2 < t1:
                    d = pltpu.make_async_copy(
                        w_hbm.at[pl.ds((t + 2) * KT, KT),
                                 pl.ds(my_pos * N_PER, N_PER)],
                        wtile_ref.at[t % 2], wdma_sems.at[t % 2])
                    d.start()
                    wd.append((t + 2, d))
                w_ref[pl.ds(t * KT, KT), :] = (
                    wtile_ref[t % 2].astype(jnp.float8_e5m2))

        w_tiles(0, 4)

        recv_cw(0, 0)
        send_cw(cw_ref.at[0, pl.ds(0, RS), :], 1, 0)
        recv_ccw(0, 0)
        send_ccw(ccw_ref.at[0, pl.ds(0, RS), :], 1, 0)

        w_tiles(4, 8)

        recv_cw(0, 1)
        send_cw(cw_ref.at[0, pl.ds(RS, RS), :], 1, 1)
        recv_ccw(0, 1)
        send_ccw(ccw_ref.at[0, pl.ds(RS, RS), :], 1, 1)

        gemm_half(my_pos, 0, x8_ref[pl.ds(0, H), :])
        gemm_half(my_pos, H, x8_ref[pl.ds(H, H), :])
        gemm_half(lax.rem(my_pos + N_DEV - 1, N_DEV), 0, cw_ref[0])

        recv_cw(1, 0)
        send_cw(cw_ref.at[1, pl.ds(0, RS), :], 2, 0)
        recv_ccw(1, 0)
        send_ccw(ccw_ref.at[1, pl.ds(0, RS), :], 2, 0)
        gemm_half(lax.rem(my_pos + 1, N_DEV), H, ccw_ref[0])
        recv_cw(1, 1)
        send_cw(cw_ref.at[1, pl.ds(RS, RS), :], 2, 1)
        recv_ccw(1, 1)
        send_ccw(ccw_ref.at[1, pl.ds(RS, RS), :], 2, 1)
        gemm_half(lax.rem(my_pos + N_DEV - 2, N_DEV), 0, cw_ref[1])
        gemm_half(lax.rem(my_pos + 2, N_DEV), H, ccw_ref[1])

        cw_org = lax.rem(my_pos + 1, N_DEV)
        ccw_org = lax.rem(my_pos + N_DEV - 1, N_DEV)
        recv_cw(2, 0)
        gemm_part(cw_org, 0, cw_ref[2, pl.ds(0, RS), :], RS)
        recv_ccw(2, 0)
        gemm_part(ccw_org, H, ccw_ref[2, pl.ds(0, RS), :], RS)
        recv_cw(2, 1)
        gemm_part(cw_org, RS, cw_ref[2, pl.ds(RS, RS), :], RS)
        recv_ccw(2, 1)
        gemm_part(ccw_org, H + RS, ccw_ref[2, pl.ds(RS, RS), :], RS)

        for r in sends:
            r.wait_send()
        for cp in cps:
            if cp is not None:
                cp.wait()

    return pl.pallas_call(
        body,
        out_shape=jax.ShapeDtypeStruct((N_DEV * M_PER, N_PER), jnp.float32),
        in_specs=[
            pl.BlockSpec(memory_space=pl.ANY),
            pl.BlockSpec(memory_space=pl.ANY),
            pl.BlockSpec(memory_space=pltpu.SMEM),
            pl.BlockSpec(memory_space=pltpu.SMEM),
        ],
        out_specs=pl.BlockSpec(memory_space=pl.ANY),
        scratch_shapes=[
            pltpu.VMEM((M_PER, K), jnp.float8_e5m2),
            pltpu.VMEM((2, RS, K), jnp.float32),
            pltpu.VMEM((K, N_PER), jnp.float8_e5m2),
            pltpu.VMEM((2, KT, N_PER), jnp.float32),
            pltpu.VMEM((3, H, K), jnp.float8_e5m2),
            pltpu.VMEM((3, H, K), jnp.float8_e5m2),
            pltpu.VMEM((N_STAGE, H, N_PER), jnp.float32),
            pltpu.SemaphoreType.DMA((2,)),
            pltpu.SemaphoreType.DMA((2,)),
            pltpu.SemaphoreType.DMA((3, 2)),
            pltpu.SemaphoreType.DMA((3, 2)),
            pltpu.SemaphoreType.DMA((3, 2)),
            pltpu.SemaphoreType.DMA((3, 2)),
            pltpu.SemaphoreType.DMA((N_STAGE,)),
        ],
        compiler_params=pltpu.CompilerParams(
            collective_id=0, vmem_limit_bytes=128 * 1024 * 1024,
        ),
    )(x, w_mat, scale_x, scale_w)
